# baseline (speedup 1.0000x reference)
import functools

import jax
import jax.numpy as jnp
from jax import lax
from jax.experimental import pallas as pl
from jax.experimental.pallas import tpu as pltpu
from jax.experimental.pallas import tpu_sc as plsc

D = 32
NNS1 = 5
B = 16384
NC = 2
NS = 16
NW = NC * NS
BPW = B // NW
QPW = BPW * 2
CPW = BPW * NNS1
CH = 128

_mesh = plsc.VectorSubcoreMesh(
    core_axis_name="c", subcore_axis_name="s", num_cores=NC, num_subcores=NS)


@functools.partial(
    pl.kernel,
    out_type=jax.ShapeDtypeStruct((B * NNS1,), jnp.float32),
    mesh=_mesh,
    compiler_params=pltpu.CompilerParams(
        needs_layout_passes=False, use_tc_tiling_on_sc=False),
    scratch_types=[
        pltpu.VMEM((QPW,), jnp.int32),
        pltpu.VMEM((CPW,), jnp.int32),
        pltpu.VMEM((QPW, D), jnp.float32),
        pltpu.VMEM((CPW, D), jnp.float32),
        pltpu.VMEM((CPW,), jnp.float32),
        pltpu.SemaphoreType.DMA,
    ],
)
def _w2v_sc(tgt_hbm, ctx_hbm, qtab_hbm, ptab_hbm, out_hbm,
            qidx_v, cidx_v, qrows_v, crows_v, out_v, sem):
    wid = lax.axis_index("s") * NC + lax.axis_index("c")

    pltpu.sync_copy(tgt_hbm.at[pl.ds(wid * QPW, QPW)], qidx_v)
    pltpu.sync_copy(ctx_hbm.at[pl.ds(wid * CPW, CPW)], cidx_v)

    copies = []
    for i in range(QPW // CH):
        copies.append(pltpu.async_copy(
            qtab_hbm.at[qidx_v.at[pl.ds(i * CH, CH)]],
            qrows_v.at[pl.ds(i * CH, CH)], sem))
    for i in range(CPW // CH):
        copies.append(pltpu.async_copy(
            ptab_hbm.at[cidx_v.at[pl.ds(i * CH, CH)]],
            crows_v.at[pl.ds(i * CH, CH)], sem))
    for cp in copies:
        cp.wait()

    iota16 = lax.iota(jnp.int32, 16)

    def gstep(g, carry):
        bvec = g * 16 + iota16
        q0 = 2 * bvec
        q1 = q0 + 1
        cix = [NNS1 * bvec + n for n in range(NNS1)]
        acc = [jnp.zeros((16,), jnp.float32) for _ in range(NNS1)]
        for d in range(D):
            dcol = jnp.full((16,), d, jnp.int32)
            w = (plsc.load_gather(qrows_v, [q0, dcol])
                 + plsc.load_gather(qrows_v, [q1, dcol]))
            for n in range(NNS1):
                acc[n] = acc[n] + plsc.load_gather(crows_v, [cix[n], dcol]) * w
        for n in range(NNS1):
            plsc.store_scatter(out_v, [cix[n]], acc[n] * 0.5)
        return carry

    lax.fori_loop(0, BPW // 16, gstep, 0)

    pltpu.sync_copy(out_v, out_hbm.at[pl.ds(wid * CPW, CPW)])


def kernel(target, context, query_table, poi_table):
    out = _w2v_sc(target.reshape(-1), context.reshape(-1),
                  query_table, poi_table)
    return out.reshape(B, NNS1)
